# baseline (speedup 1.0000x reference)
import functools

import jax
import jax.numpy as jnp
from jax import lax
from jax.experimental import pallas as pl
from jax.experimental.pallas import tpu as pltpu
from jax.experimental.pallas import tpu_sc as plsc

_BATCH = 16384
_D = 32
_NC = 2
_NS = 16
_L = 16
_NW = _NC * _NS
_BPW = _BATCH // _NW
_G = _BPW // _L

_mesh = plsc.VectorSubcoreMesh(core_axis_name="c", subcore_axis_name="s")


@functools.partial(
    pl.kernel,
    out_type=jax.ShapeDtypeStruct((_BATCH,), jnp.float32),
    mesh=_mesh,
    scratch_types=[
        pltpu.VMEM((_BPW,), jnp.int32),
        pltpu.VMEM((_BPW,), jnp.int32),
        pltpu.VMEM((_D, _BPW), jnp.float32),
        pltpu.VMEM((_D, _BPW), jnp.float32),
        pltpu.VMEM((_BPW,), jnp.float32),
        pltpu.VMEM((_BPW,), jnp.float32),
        pltpu.VMEM((_L,), jnp.float32),
        pltpu.VMEM((_BPW,), jnp.float32),
        pltpu.SemaphoreType.DMA,
    ],
    compiler_params=pltpu.CompilerParams(needs_layout_passes=False,
                                         use_tc_tiling_on_sc=False),
)
def _mf_score(u_ids, i_ids, uT, iT, u_bias, i_bias, bias16, out,
         uidx_v, iidx_v, ucols, icols, ub_v, ib_v, b_v, out_v, sem):
    wid = lax.axis_index("s") * _NC + lax.axis_index("c")
    base = wid * _BPW

    pltpu.sync_copy(u_ids.at[pl.ds(base, _BPW)], uidx_v)
    pltpu.sync_copy(i_ids.at[pl.ds(base, _BPW)], iidx_v)
    pltpu.sync_copy(bias16, b_v)

    copies = []
    copies.append(pltpu.async_copy(u_bias.at[uidx_v], ub_v, sem))
    copies.append(pltpu.async_copy(i_bias.at[iidx_v], ib_v, sem))
    for d in range(_D):
        copies.append(pltpu.async_copy(uT.at[d].at[uidx_v], ucols.at[d], sem))
        copies.append(pltpu.async_copy(iT.at[d].at[iidx_v], icols.at[d], sem))
    for c in copies:
        c.wait()

    b_vec = b_v[pl.ds(0, _L)]

    def group(g, carry):
        s0 = g * _L
        acc = b_vec + ub_v[pl.ds(s0, _L)] + ib_v[pl.ds(s0, _L)]
        for d in range(_D):
            acc = acc + ucols[d, pl.ds(s0, _L)] * icols[d, pl.ds(s0, _L)]
        out_v[pl.ds(s0, _L)] = acc
        return carry

    lax.fori_loop(0, _G, group, 0)
    pltpu.sync_copy(out_v, out.at[pl.ds(base, _BPW)])


def kernel(u_ids, i_ids, user_embeddings, item_embeddings,
            user_bias, item_bias, bias):
    bias16 = jnp.broadcast_to(jnp.reshape(bias, (1,)), (_L,))
    return _mf_score(u_ids, i_ids, user_embeddings.T, item_embeddings.T,
                user_bias, item_bias, bias16)


# speedup vs baseline: 4.8674x; 4.8674x over previous
import functools

import jax
import jax.numpy as jnp
from jax import lax
from jax.experimental import pallas as pl
from jax.experimental.pallas import tpu as pltpu
from jax.experimental.pallas import tpu_sc as plsc

_BATCH = 16384
_D = 32
_NC = 2
_NS = 16
_L = 16
_NW = _NC * _NS
_BPW = _BATCH // _NW
_G = _BPW // _L

_mesh = plsc.VectorSubcoreMesh(core_axis_name="c", subcore_axis_name="s")


@functools.partial(
    pl.kernel,
    out_type=jax.ShapeDtypeStruct((_BATCH,), jnp.float32),
    mesh=_mesh,
    scratch_types=[
        pltpu.VMEM((_BPW,), jnp.int32),
        pltpu.VMEM((_BPW,), jnp.int32),
        pltpu.VMEM((_BPW, _D), jnp.bfloat16),
        pltpu.VMEM((_BPW, _D), jnp.bfloat16),
        pltpu.VMEM((_BPW,), jnp.float32),
        pltpu.VMEM((_BPW,), jnp.float32),
        pltpu.VMEM((_L,), jnp.float32),
        pltpu.VMEM((_L * _L,), jnp.float32),
        pltpu.VMEM((_BPW,), jnp.float32),
        pltpu.SemaphoreType.DMA,
    ],
    compiler_params=pltpu.CompilerParams(needs_layout_passes=False,
                                         use_tc_tiling_on_sc=False),
)
def _mf_score(u_ids, i_ids, u_emb, i_emb, u_bias, i_bias, bias16, out,
              uidx_v, iidx_v, urows, irows, ub_v, ib_v, b_v, pT, out_v, sem):
    wid = lax.axis_index("s") * _NC + lax.axis_index("c")
    base = wid * _BPW

    pltpu.sync_copy(u_ids.at[pl.ds(base, _BPW)], uidx_v)
    pltpu.sync_copy(i_ids.at[pl.ds(base, _BPW)], iidx_v)
    pltpu.sync_copy(bias16, b_v)

    c1 = pltpu.async_copy(u_emb.at[uidx_v], urows, sem)
    c2 = pltpu.async_copy(i_emb.at[iidx_v], irows, sem)
    c3 = pltpu.async_copy(u_bias.at[uidx_v], ub_v, sem)
    c4 = pltpu.async_copy(i_bias.at[iidx_v], ib_v, sem)
    c1.wait()
    c2.wait()
    c3.wait()
    c4.wait()

    lanes16 = lax.iota(jnp.int32, 16) * _L
    b_vec = b_v[pl.ds(0, _L)]

    def group(g, carry):
        s0 = g * _L
        for sl in range(_L):
            s = s0 + sl
            ua, ub = plsc.unpack(urows[s, pl.ds(0, _D)],
                                 format=plsc.PackFormat.INTERLEAVED)
            ia, ib = plsc.unpack(irows[s, pl.ds(0, _D)],
                                 format=plsc.PackFormat.INTERLEAVED)
            p = ua * ia + ub * ib
            plsc.store_scatter(pT, [lanes16 + sl, ], p)
        acc = b_vec
        for d in range(_L):
            acc = acc + pT[pl.ds(d * _L, _L)]
        acc = acc + ub_v[pl.ds(s0, _L)] + ib_v[pl.ds(s0, _L)]
        out_v[pl.ds(s0, _L)] = acc
        return carry

    lax.fori_loop(0, _G, group, 0)
    pltpu.sync_copy(out_v, out.at[pl.ds(base, _BPW)])


def kernel(u_ids, i_ids, user_embeddings, item_embeddings,
           user_bias, item_bias, bias):
    bias16 = jnp.broadcast_to(jnp.reshape(bias, (1,)), (_L,))
    return _mf_score(u_ids, i_ids,
                     user_embeddings.astype(jnp.bfloat16),
                     item_embeddings.astype(jnp.bfloat16),
                     user_bias, item_bias, bias16)


# speedup vs baseline: 21.7375x; 4.4660x over previous
import functools

import jax
import jax.numpy as jnp
from jax import lax
from jax.experimental import pallas as pl
from jax.experimental.pallas import tpu as pltpu
from jax.experimental.pallas import tpu_sc as plsc

_BATCH = 16384
_D = 32
_NC = 2
_NS = 16
_L = 16
_NW = _NC * _NS
_BPW = _BATCH // _NW
_G = _BPW // _L
_RING = 8

_mesh = plsc.VectorSubcoreMesh(core_axis_name="c", subcore_axis_name="s")


def _extract(ids_ref, j, lanes):
    g16 = (j // _L) * _L
    idv = ids_ref[pl.ds(g16, _L)]
    sel = jnp.where(lanes == (j % _L), idv, jnp.zeros((_L,), jnp.int32))
    return jnp.sum(sel)


@functools.partial(
    pl.kernel,
    out_type=jax.ShapeDtypeStruct((_BATCH,), jnp.float32),
    mesh=_mesh,
    scratch_types=[
        pltpu.VMEM((_BPW,), jnp.int32),
        pltpu.VMEM((_BPW,), jnp.int32),
        pltpu.VMEM((_RING, _D, 128), jnp.float32),
        pltpu.VMEM((_RING, _D, 128), jnp.float32),
        pltpu.VMEM((_L * _L,), jnp.float32),
        pltpu.VMEM((_BPW,), jnp.float32),
        [pltpu.SemaphoreType.DMA] * _RING,
        [pltpu.SemaphoreType.DMA] * _RING,
    ],
    compiler_params=pltpu.CompilerParams(needs_layout_passes=False,
                                         use_tc_tiling_on_sc=True),
)
def _mf3(u_ids, i_ids, uT, iT, out,
         uidx_v, iidx_v, ubufs, ibufs, pT, out_v, usems, isems):
    wid = lax.axis_index("s") * _NC + lax.axis_index("c")
    base = wid * _BPW

    pltpu.sync_copy(u_ids.at[pl.ds(base, _BPW)], uidx_v)
    pltpu.sync_copy(i_ids.at[pl.ds(base, _BPW)], iidx_v)

    lanes = lax.iota(jnp.int32, _L)
    lanes16 = lanes * _L

    def fire(j, k):
        ur = _extract(uidx_v, j, lanes)
        ir = _extract(iidx_v, j, lanes)
        uc = pl.multiple_of((ur >> 7) << 7, 128)
        ic = pl.multiple_of((ir >> 7) << 7, 128)
        pltpu.async_copy(uT.at[:, pl.ds(uc, 128)], ubufs.at[k], usems[k])
        pltpu.async_copy(iT.at[:, pl.ds(ic, 128)], ibufs.at[k], isems[k])

    for k in range(_RING):
        fire(k, k)

    def body(j, carry):
        for k in range(_RING):
            @pl.when(j % _RING == k)
            def _():
                pltpu.make_async_copy(
                    uT.at[:, pl.ds(0, 128)], ubufs.at[k], usems[k]).wait()
                pltpu.make_async_copy(
                    iT.at[:, pl.ds(0, 128)], ibufs.at[k], isems[k]).wait()
                ur = _extract(uidx_v, j, lanes)
                ir = _extract(iidx_v, j, lanes)
                ul = jnp.full((_L,), ur & 127, jnp.int32)
                il = jnp.full((_L,), ir & 127, jnp.int32)
                u0 = plsc.load_gather(ubufs.at[k], [lanes, ul])
                u1 = plsc.load_gather(ubufs.at[k], [lanes + _L, ul])
                i0 = plsc.load_gather(ibufs.at[k], [lanes, il])
                i1 = plsc.load_gather(ibufs.at[k], [lanes + _L, il])
                p = u0 * i0 + u1 * i1
                plsc.store_scatter(pT, [lanes16 + (j % _L)], p)

                @pl.when(j < _BPW - _RING)
                def _():
                    fire(j + _RING, k)

        @pl.when(j % _L == _L - 1)
        def _():
            acc = jnp.zeros((_L,), jnp.float32)
            for d in range(_L):
                acc = acc + pT[pl.ds(d * _L, _L)]
            out_v[pl.ds((j // _L) * _L, _L)] = acc

        return carry

    lax.fori_loop(0, _BPW, body, 0)
    pltpu.sync_copy(out_v, out.at[pl.ds(base, _BPW)])


@functools.partial(
    pl.kernel,
    out_type=jax.ShapeDtypeStruct((_BATCH,), jnp.float32),
    mesh=_mesh,
    scratch_types=[
        pltpu.VMEM((_BPW,), jnp.int32),
        pltpu.VMEM((_BPW,), jnp.int32),
        pltpu.VMEM((_BPW,), jnp.float32),
        pltpu.VMEM((_BPW,), jnp.float32),
        pltpu.VMEM((_L,), jnp.float32),
        pltpu.VMEM((_BPW,), jnp.float32),
        pltpu.SemaphoreType.DMA,
    ],
    compiler_params=pltpu.CompilerParams(needs_layout_passes=False,
                                         use_tc_tiling_on_sc=False),
)
def _bias3(u_ids, i_ids, u_bias, i_bias, bias16, out,
           uidx_v, iidx_v, ub_v, ib_v, b_v, out_v, sem):
    wid = lax.axis_index("s") * _NC + lax.axis_index("c")
    base = wid * _BPW
    pltpu.sync_copy(u_ids.at[pl.ds(base, _BPW)], uidx_v)
    pltpu.sync_copy(i_ids.at[pl.ds(base, _BPW)], iidx_v)
    pltpu.sync_copy(bias16, b_v)
    c1 = pltpu.async_copy(u_bias.at[uidx_v], ub_v, sem)
    c2 = pltpu.async_copy(i_bias.at[iidx_v], ib_v, sem)
    c1.wait()
    c2.wait()
    b_vec = b_v[pl.ds(0, _L)]

    def group(g, carry):
        s0 = g * _L
        out_v[pl.ds(s0, _L)] = (b_vec + ub_v[pl.ds(s0, _L)]
                                + ib_v[pl.ds(s0, _L)])
        return carry

    lax.fori_loop(0, _G, group, 0)
    pltpu.sync_copy(out_v, out.at[pl.ds(base, _BPW)])


def kernel(u_ids, i_ids, user_embeddings, item_embeddings,
            user_bias, item_bias, bias):
    bias16 = jnp.broadcast_to(jnp.reshape(bias, (1,)), (_L,))
    dots = _mf3(u_ids, i_ids, user_embeddings.T, item_embeddings.T)
    part = _bias3(u_ids, i_ids, user_bias, item_bias, bias16)
    return dots + part
